# baseline (speedup 1.0000x reference)
import functools

import jax
import jax.numpy as jnp
from jax import lax
from jax.experimental import pallas as pl
from jax.experimental.pallas import tpu as pltpu
from jax.experimental.pallas import tpu_sc as plsc

NC = 2
NS = 16
NW = NC * NS

K = 128
GPG = 2
ROWS_G = K * GPG
W = 128


@functools.partial(jax.jit, static_argnames=("b", "d"))
def _gather_rows(xf, tableP, b, d):
    n_per_w = b // NW
    n_groups = n_per_w // ROWS_G
    mesh = plsc.VectorSubcoreMesh(core_axis_name="c", subcore_axis_name="s")

    @functools.partial(
        pl.kernel,
        out_type=jax.ShapeDtypeStruct((b, W), jnp.float32),
        mesh=mesh,
        scratch_types=[
            pltpu.VMEM((n_per_w,), jnp.float32),
            pltpu.VMEM((n_per_w,), jnp.int32),
            pltpu.VMEM((ROWS_G, W), jnp.float32),
            pltpu.VMEM((ROWS_G, W), jnp.float32),
            pltpu.SemaphoreType.DMA,
            pltpu.SemaphoreType.DMA,
            pltpu.SemaphoreType.DMA,
            pltpu.SemaphoreType.DMA,
        ],
        compiler_params=pltpu.CompilerParams(use_tc_tiling_on_sc=False),
    )
    def k(xf_hbm, table_hbm, out_hbm, idxf_v, idx_v, buf0, buf1,
          gsem0, gsem1, osem0, osem1):
        bufs = (buf0, buf1)
        gsems = (gsem0, gsem1)
        osems = (osem0, osem1)
        wid = lax.axis_index("s") * NC + lax.axis_index("c")
        pltpu.sync_copy(xf_hbm.at[pl.ds(wid * n_per_w, n_per_w)], idxf_v)

        def conv_chunk(r, c):
            for j in range(8):
                o = r * 128 + j * 16
                idx_v[pl.ds(o, 16)] = idxf_v[pl.ds(o, 16)].astype(jnp.int32)
            return c

        lax.fori_loop(0, n_per_w // 128, conv_chunk, 0)

        out_base = wid * n_per_w

        def fire_gather(g, bi):
            for j in range(GPG):
                pltpu.async_copy(
                    table_hbm.at[idx_v.at[pl.ds((g * GPG + j) * K, K)]],
                    bufs[bi].at[pl.ds(j * K, K)],
                    gsems[bi],
                )

        def wait_gather(bi):
            pltpu.make_async_copy(
                table_hbm.at[pl.ds(0, ROWS_G)], bufs[bi], gsems[bi]
            ).wait()

        def fire_out(g, bi):
            pltpu.async_copy(
                bufs[bi],
                out_hbm.at[pl.ds(out_base + g * ROWS_G, ROWS_G)],
                osems[bi],
            )

        def wait_out(bi):
            pltpu.make_async_copy(
                bufs[bi], out_hbm.at[pl.ds(0, ROWS_G)], osems[bi]
            ).wait()

        fire_gather(0, 0)
        wait_gather(0)
        fire_out(0, 0)
        fire_gather(1, 1)

        def body(i, c):
            for bi, off in ((1, 1), (0, 2)):
                g = 2 * i + off
                wait_gather(bi)
                fire_out(g, bi)
                wait_out(1 - bi)
                fire_gather(g + 1, 1 - bi)
            return c

        lax.fori_loop(0, (n_groups - 2) // 2, body, 0)

        wait_gather(1)
        fire_out(n_groups - 1, 1)
        wait_out(0)
        wait_out(1)

    return k(xf, tableP)


def kernel(x, table):
    b, t = x.shape
    d = table.shape[1]
    xf = x.reshape(-1).astype(jnp.float32)
    tableP = jnp.pad(table, ((0, 0), (0, W - d)))
    out = _gather_rows(xf, tableP, b * t, d)
    return out[:, :d].reshape(b, t, d)


# speedup vs baseline: 1.0861x; 1.0861x over previous
import functools

import jax
import jax.numpy as jnp
from jax import lax
from jax.experimental import pallas as pl
from jax.experimental.pallas import tpu as pltpu
from jax.experimental.pallas import tpu_sc as plsc

NC = 2
NS = 16
NW = NC * NS

K = 128
GPG = 4
ROWS_G = K * GPG
W = 128


@functools.partial(jax.jit, static_argnames=("b", "d"))
def _gather_rows(xf, table, b, d):
    n_per_w = b // NW
    n_groups = n_per_w // ROWS_G
    mesh = plsc.VectorSubcoreMesh(core_axis_name="c", subcore_axis_name="s")

    @functools.partial(
        pl.kernel,
        out_type=jax.ShapeDtypeStruct((b, W), jnp.float32),
        mesh=mesh,
        scratch_types=[
            pltpu.VMEM((n_per_w,), jnp.float32),
            pltpu.VMEM((n_per_w,), jnp.int32),
            pltpu.VMEM((ROWS_G, d), jnp.float32),
            pltpu.VMEM((ROWS_G, d), jnp.float32),
            pltpu.SemaphoreType.DMA,
            pltpu.SemaphoreType.DMA,
            pltpu.SemaphoreType.DMA,
            pltpu.SemaphoreType.DMA,
        ],
        compiler_params=pltpu.CompilerParams(use_tc_tiling_on_sc=False),
    )
    def k(xf_hbm, table_hbm, out_hbm, idxf_v, idx_v, buf0, buf1,
          gsem0, gsem1, osem0, osem1):
        bufs = (buf0, buf1)
        gsems = (gsem0, gsem1)
        osems = (osem0, osem1)
        wid = lax.axis_index("s") * NC + lax.axis_index("c")
        pltpu.sync_copy(xf_hbm.at[pl.ds(wid * n_per_w, n_per_w)], idxf_v)

        def conv_chunk(r, c):
            for j in range(8):
                o = r * 128 + j * 16
                idx_v[pl.ds(o, 16)] = idxf_v[pl.ds(o, 16)].astype(jnp.int32)
            return c

        lax.fori_loop(0, n_per_w // 128, conv_chunk, 0)

        out_base = wid * n_per_w

        def fire_gather(g, bi):
            for j in range(GPG):
                pltpu.async_copy(
                    table_hbm.at[idx_v.at[pl.ds((g * GPG + j) * K, K)]],
                    bufs[bi].at[pl.ds(j * K, K)],
                    gsems[bi],
                )

        def wait_gather(bi):
            pltpu.make_async_copy(
                table_hbm.at[pl.ds(0, ROWS_G)], bufs[bi], gsems[bi]
            ).wait()

        def fire_out(g, bi):
            pltpu.async_copy(
                bufs[bi],
                out_hbm.at[pl.ds(out_base + g * ROWS_G, ROWS_G), pl.ds(0, d)],
                osems[bi],
            )

        def wait_out(bi):
            pltpu.make_async_copy(
                bufs[bi], out_hbm.at[pl.ds(0, ROWS_G), pl.ds(0, d)], osems[bi]
            ).wait()

        fire_gather(0, 0)
        wait_gather(0)
        fire_out(0, 0)
        fire_gather(1, 1)

        def body(i, c):
            for bi, off in ((1, 1), (0, 2)):
                g = 2 * i + off
                wait_gather(bi)
                fire_out(g, bi)
                wait_out(1 - bi)
                fire_gather(g + 1, 1 - bi)
            return c

        lax.fori_loop(0, (n_groups - 2) // 2, body, 0)

        wait_gather(1)
        fire_out(n_groups - 1, 1)
        wait_out(0)
        wait_out(1)

    return k(xf, table)


def kernel(x, table):
    b, t = x.shape
    d = table.shape[1]
    xf = x.reshape(-1).astype(jnp.float32)
    out = _gather_rows(xf, table, b * t, d)
    return out[:, :d].reshape(b, t, d)


# speedup vs baseline: 1.0899x; 1.0035x over previous
import functools

import jax
import jax.numpy as jnp
from jax import lax
from jax.experimental import pallas as pl
from jax.experimental.pallas import tpu as pltpu
from jax.experimental.pallas import tpu_sc as plsc

NC = 2
NS = 16
NW = NC * NS

K = 128
GPG = 4
ROWS_G = K * GPG
W = 128


@functools.partial(jax.jit, static_argnames=("b", "d"))
def _gather_rows(xf, table, b, d):
    n_per_w = b // NW
    n_groups = n_per_w // ROWS_G
    mesh = plsc.VectorSubcoreMesh(core_axis_name="c", subcore_axis_name="s")

    @functools.partial(
        pl.kernel,
        out_type=jax.ShapeDtypeStruct((b, W), jnp.float32),
        mesh=mesh,
        scratch_types=[
            pltpu.VMEM((n_per_w,), jnp.float32),
            pltpu.VMEM((n_per_w,), jnp.int32),
            pltpu.VMEM((ROWS_G, d), jnp.float32),
            pltpu.VMEM((ROWS_G, d), jnp.float32),
            pltpu.SemaphoreType.DMA,
            pltpu.SemaphoreType.DMA,
            pltpu.SemaphoreType.DMA,
            pltpu.SemaphoreType.DMA,
        ],
        compiler_params=pltpu.CompilerParams(use_tc_tiling_on_sc=False),
    )
    def k(xf_hbm, table_hbm, out_hbm, idxf_v, idx_v, buf0, buf1,
          gsem0, gsem1, osem0, osem1):
        bufs = (buf0, buf1)
        gsems = (gsem0, gsem1)
        osems = (osem0, osem1)
        wid = lax.axis_index("s") * NC + lax.axis_index("c")
        pltpu.sync_copy(xf_hbm.at[pl.ds(wid * n_per_w, n_per_w)], idxf_v)

        def conv_chunk(r, c):
            for j in range(8):
                o = r * 128 + j * 16
                idx_v[pl.ds(o, 16)] = idxf_v[pl.ds(o, 16)].astype(jnp.int32)
            return c

        out_base = wid * n_per_w

        def fire_gather(g, bi):
            for j in range(GPG):
                pltpu.async_copy(
                    table_hbm.at[idx_v.at[pl.ds((g * GPG + j) * K, K)]],
                    bufs[bi].at[pl.ds(j * K, K)],
                    gsems[bi],
                )

        def wait_gather(bi):
            pltpu.make_async_copy(
                table_hbm.at[pl.ds(0, ROWS_G)], bufs[bi], gsems[bi]
            ).wait()

        def fire_out(g, bi):
            pltpu.async_copy(
                bufs[bi],
                out_hbm.at[pl.ds(out_base + g * ROWS_G, ROWS_G), pl.ds(0, d)],
                osems[bi],
            )

        def wait_out(bi):
            pltpu.make_async_copy(
                bufs[bi], out_hbm.at[pl.ds(0, ROWS_G), pl.ds(0, d)], osems[bi]
            ).wait()

        lax.fori_loop(0, 2 * ROWS_G // 128, conv_chunk, 0)
        fire_gather(0, 0)
        fire_gather(1, 1)
        lax.fori_loop(2 * ROWS_G // 128, n_per_w // 128, conv_chunk, 0)
        wait_gather(0)
        fire_out(0, 0)

        def body(i, c):
            for bi, off in ((1, 1), (0, 2)):
                g = 2 * i + off
                wait_gather(bi)
                wait_out(1 - bi)
                fire_gather(g + 1, 1 - bi)
                fire_out(g, bi)
            return c

        lax.fori_loop(0, (n_groups - 2) // 2, body, 0)

        wait_gather(1)
        wait_out(0)
        fire_out(n_groups - 1, 1)
        wait_out(1)

    return k(xf, table)


def kernel(x, table):
    b, t = x.shape
    d = table.shape[1]
    xf = x.reshape(-1).astype(jnp.float32)
    out = _gather_rows(xf, table, b * t, d)
    return out[:, :d].reshape(b, t, d)
